# baseline (speedup 1.0000x reference)
import jax
import jax.numpy as jnp
from jax import lax
from jax.experimental import pallas as pl
from jax.experimental.pallas import tpu as pltpu
import jax.experimental.pallas.tpu_sc as plsc

NR = 320000
D = 128
S = 10000
NC = 2
NS = 16
NW = NC * NS
RPT = NR // NW
B = 80
NB = RPT // B
SP = 10240
CH = SP // NS
TRASH = S + 200


def _sc_body(x_hbm, ids_hbm, sums_hbm, counts_hbm,
             ids_v, xbuf0, xbuf1, ones_v, zcnt_v, eid_v, ebuf, cbuf, idxb_v,
             idxc_v, sem0, sem1, ssem0, ssem1, csem0, csem1, acc_sh, cnt_sh):
    c = lax.axis_index("c")
    s = lax.axis_index("s")
    w = c * NS + s

    zeros16 = jnp.zeros((16,), jnp.float32)
    for k in range(B // 16):
        ones_v[pl.ds(k * 16, 16)] = jnp.ones((16,), jnp.float32)

    def zrow(i, carry):
        for k in range(D // 16):
            xbuf0[i, pl.ds(k * 16, 16)] = zeros16
        return carry

    lax.fori_loop(0, B, zrow, 0)

    def zc(i, carry):
        zcnt_v[pl.ds(i * 16, 16)] = zeros16
        return carry

    lax.fori_loop(0, CH // 16, zc, 0)

    for k in range(CH // B):
        pltpu.sync_copy(xbuf0, acc_sh.at[pl.ds(s * CH + k * B, B), :])
    pltpu.sync_copy(zcnt_v, cnt_sh.at[pl.ds(s * CH, CH)])
    plsc.subcore_barrier()

    pltpu.sync_copy(ids_hbm.at[w], ids_v)
    last_id = ids_v[NB - 1, pl.ds(B - 16, 16)][15]
    first_id = ids_v[0, pl.ds(0, 16)][0]
    p0 = S + 8 * s
    p1 = p0 + 1

    base = w * RPT

    def dma(j, buf, sem):
        row = base + j * B
        return pltpu.make_async_copy(x_hbm.at[pl.ds(row, B), :], buf, sem)

    def route(j, idxb):
        for k in range(B // 16):
            v = ids_v[j, pl.ds(k * 16, 16)]
            idxb[pl.ds(k * 16, 16)] = jnp.where(
                v == first_id, p0, jnp.where(v == last_id, p1, v))

    def scat_start(buf, idxb, ssem, csem):
        pltpu.async_copy(buf, acc_sh.at[idxb], ssem, add=True)
        pltpu.async_copy(ones_v, cnt_sh.at[idxb], csem, add=True)

    def scat_wait(buf, idxb, ssem, csem):
        pltpu.make_async_copy(buf, acc_sh.at[idxb], ssem).wait()
        pltpu.make_async_copy(ones_v, cnt_sh.at[idxb], csem).wait()

    dma(0, xbuf0, sem0).start()
    dma(1, xbuf1, sem1).start()

    def pair(g, carry):
        j0 = 2 * g
        j1 = 2 * g + 1
        dma(j0, xbuf0, sem0).wait()
        route(j0, idxb_v)
        scat_start(xbuf0, idxb_v, ssem0, csem0)
        dma(j1, xbuf1, sem1).wait()
        route(j1, idxc_v)
        scat_start(xbuf1, idxc_v, ssem1, csem1)
        scat_wait(xbuf0, idxb_v, ssem0, csem0)
        dma(j0 + 2, xbuf0, sem0).start()
        scat_wait(xbuf1, idxc_v, ssem1, csem1)
        dma(jnp.minimum(j1 + 2, NB - 1), xbuf1, sem1).start()
        return carry

    lax.fori_loop(0, (NB - 1) // 2, pair, 0)

    dma(NB - 1, xbuf0, sem0).wait()
    route(NB - 1, idxb_v)
    scat_start(xbuf0, idxb_v, ssem0, csem0)
    scat_wait(xbuf0, idxb_v, ssem0, csem0)
    dma(NB - 1, xbuf1, sem1).wait()
    plsc.subcore_barrier()

    k16 = lax.broadcasted_iota(jnp.int32, (16,), 0)

    @pl.when(s == 0)
    def _():
        for t in range(NS):
            w2 = c * NS + t
            pltpu.sync_copy(ids_hbm.at[w2, 0, pl.ds(0, 16)], eid_v)
            f = eid_v[...][0]
            pltpu.sync_copy(ids_hbm.at[w2, NB - 1, pl.ds(B - 16, 16)], eid_v)
            l = eid_v[...][15]
            idx = jnp.where(k16 == 0, f, jnp.where(k16 == 1, l, TRASH))
            pltpu.sync_copy(acc_sh.at[pl.ds(S + 8 * t, 16), :], ebuf)
            pltpu.sync_copy(ebuf, acc_sh.at[idx], add=True)
            pltpu.sync_copy(cnt_sh.at[pl.ds(S + 8 * t, 16)], cbuf)
            pltpu.sync_copy(cbuf, cnt_sh.at[idx], add=True)

    plsc.subcore_barrier()

    for k in range(CH // B):
        r0 = s * CH + k * B
        pltpu.sync_copy(acc_sh.at[pl.ds(r0, B), :], xbuf1)
        pltpu.sync_copy(xbuf1, sums_hbm.at[c, pl.ds(r0, B), :])
    pltpu.sync_copy(cnt_sh.at[pl.ds(s * CH, CH)], zcnt_v)
    pltpu.sync_copy(zcnt_v, counts_hbm.at[pl.ds(c * SP + s * CH, CH)])


def _sc_segment_sum(x, ids3):
    f = pl.kernel(
        _sc_body,
        out_type=(jax.ShapeDtypeStruct((NC, SP, D), jnp.float32),
                  jax.ShapeDtypeStruct((NC * SP,), jnp.float32)),
        mesh=plsc.VectorSubcoreMesh(core_axis_name="c", subcore_axis_name="s"),
        scratch_types=(
            pltpu.VMEM((NB, B), jnp.int32),
            pltpu.VMEM((B, D), jnp.float32),
            pltpu.VMEM((B, D), jnp.float32),
            pltpu.VMEM((B,), jnp.float32),
            pltpu.VMEM((CH,), jnp.float32),
            pltpu.VMEM((16,), jnp.int32),
            pltpu.VMEM((16, D), jnp.float32),
            pltpu.VMEM((16,), jnp.float32),
            pltpu.VMEM((B,), jnp.int32),
            pltpu.VMEM((B,), jnp.int32),
            pltpu.SemaphoreType.DMA,
            pltpu.SemaphoreType.DMA,
            pltpu.SemaphoreType.DMA,
            pltpu.SemaphoreType.DMA,
            pltpu.SemaphoreType.DMA,
            pltpu.SemaphoreType.DMA,
            pltpu.VMEM_SHARED((SP, D), jnp.float32),
            pltpu.VMEM_SHARED((NS * CH,), jnp.float32),
        ),
    )
    return f(x, ids3)


def _finish_body(sums_ref, counts_ref, out_ref):
    sm = sums_ref[0, :S] + sums_ref[1, :S]
    ct = counts_ref[0] + counts_ref[1]
    nodes = sm / jnp.maximum(ct, 1.0)
    isn = jnp.isnan(nodes)
    n_ok = jnp.sum(jnp.where(isn, 0.0, 1.0))
    mean_val = jnp.sum(jnp.where(isn, 0.0, nodes)) / n_ok
    out_ref[...] = jnp.where(isn, mean_val, nodes)


def kernel(x, segment_ids):
    ids3 = segment_ids.astype(jnp.int32).reshape(NW, NB, B)
    sums_p, counts_p = _sc_segment_sum(x, ids3)
    counts3 = counts_p.reshape(NC, SP)[:, :S].reshape(NC, S, 1)
    return pl.pallas_call(
        _finish_body,
        out_shape=jax.ShapeDtypeStruct((S, D), jnp.float32),
    )(sums_p, counts3)


# speedup vs baseline: 1.0816x; 1.0816x over previous
import jax
import jax.numpy as jnp
from jax import lax
from jax.experimental import pallas as pl
from jax.experimental.pallas import tpu as pltpu
import jax.experimental.pallas.tpu_sc as plsc

NR = 320000
D = 128
S = 10000
NC = 2
NS = 16
NW = NC * NS
RPT = NR // NW
B = 80
NB = RPT // B
SP = 10240
CH = SP // NS
TRASH = S + 200


def _sc_body(x_hbm, ids_hbm, sums_hbm, counts_hbm,
             ids_v, xbuf0, xbuf1, ones_v, zcnt_v, eid_v, ebuf, cbuf, idxb_v,
             sem0, sem1, acc_sh, cnt_sh):
    c = lax.axis_index("c")
    s = lax.axis_index("s")
    w = c * NS + s

    zeros16 = jnp.zeros((16,), jnp.float32)
    for k in range(B // 16):
        ones_v[pl.ds(k * 16, 16)] = jnp.ones((16,), jnp.float32)

    def zrow(i, carry):
        for k in range(D // 16):
            xbuf0[i, pl.ds(k * 16, 16)] = zeros16
        return carry

    lax.fori_loop(0, B, zrow, 0)

    def zc(i, carry):
        zcnt_v[pl.ds(i * 16, 16)] = zeros16
        return carry

    lax.fori_loop(0, CH // 16, zc, 0)

    for k in range(CH // B):
        pltpu.sync_copy(xbuf0, acc_sh.at[pl.ds(s * CH + k * B, B), :])
    pltpu.sync_copy(zcnt_v, cnt_sh.at[pl.ds(s * CH, CH)])
    plsc.subcore_barrier()

    pltpu.sync_copy(ids_hbm.at[w], ids_v)
    last_id = ids_v[NB - 1, pl.ds(B - 16, 16)][15]
    first_id = ids_v[0, pl.ds(0, 16)][0]
    p0 = S + 8 * s
    p1 = p0 + 1

    base = w * RPT

    def dma(j, buf, sem):
        row = base + j * B
        return pltpu.make_async_copy(x_hbm.at[pl.ds(row, B), :], buf, sem)

    def route(j):
        for k in range(B // 16):
            v = ids_v[j, pl.ds(k * 16, 16)]
            idxb_v[pl.ds(k * 16, 16)] = jnp.where(
                v == first_id, p0, jnp.where(v == last_id, p1, v))

    def scat(buf):
        pltpu.sync_copy(buf, acc_sh.at[idxb_v], add=True)
        pltpu.sync_copy(ones_v, cnt_sh.at[idxb_v], add=True)

    dma(0, xbuf0, sem0).start()
    dma(1, xbuf1, sem1).start()

    def pair(g, carry):
        j0 = 2 * g
        j1 = 2 * g + 1
        dma(j0, xbuf0, sem0).wait()
        route(j0)
        scat(xbuf0)
        dma(j0 + 2, xbuf0, sem0).start()
        dma(j1, xbuf1, sem1).wait()
        route(j1)
        scat(xbuf1)
        dma(jnp.minimum(j1 + 2, NB - 1), xbuf1, sem1).start()
        return carry

    lax.fori_loop(0, (NB - 1) // 2, pair, 0)

    dma(NB - 1, xbuf0, sem0).wait()
    route(NB - 1)
    scat(xbuf0)
    dma(NB - 1, xbuf1, sem1).wait()
    plsc.subcore_barrier()

    k16 = lax.broadcasted_iota(jnp.int32, (16,), 0)

    @pl.when(s == 0)
    def _():
        for t in range(NS):
            w2 = c * NS + t
            pltpu.sync_copy(ids_hbm.at[w2, 0, pl.ds(0, 16)], eid_v)
            f = eid_v[...][0]
            pltpu.sync_copy(ids_hbm.at[w2, NB - 1, pl.ds(B - 16, 16)], eid_v)
            l = eid_v[...][15]
            idx = jnp.where(k16 == 0, f, jnp.where(k16 == 1, l, TRASH))
            pltpu.sync_copy(acc_sh.at[pl.ds(S + 8 * t, 16), :], ebuf)
            pltpu.sync_copy(ebuf, acc_sh.at[idx], add=True)
            pltpu.sync_copy(cnt_sh.at[pl.ds(S + 8 * t, 16)], cbuf)
            pltpu.sync_copy(cbuf, cnt_sh.at[idx], add=True)

    plsc.subcore_barrier()

    for k in range(CH // B):
        r0 = s * CH + k * B
        pltpu.sync_copy(acc_sh.at[pl.ds(r0, B), :], xbuf1)
        pltpu.sync_copy(xbuf1, sums_hbm.at[c, pl.ds(r0, B), :])
    pltpu.sync_copy(cnt_sh.at[pl.ds(s * CH, CH)], zcnt_v)
    pltpu.sync_copy(zcnt_v, counts_hbm.at[pl.ds(c * SP + s * CH, CH)])


def _sc_segment_sum(x, ids3):
    f = pl.kernel(
        _sc_body,
        out_type=(jax.ShapeDtypeStruct((NC, SP, D), jnp.float32),
                  jax.ShapeDtypeStruct((NC * SP,), jnp.float32)),
        mesh=plsc.VectorSubcoreMesh(core_axis_name="c", subcore_axis_name="s"),
        scratch_types=(
            pltpu.VMEM((NB, B), jnp.int32),
            pltpu.VMEM((B, D), jnp.float32),
            pltpu.VMEM((B, D), jnp.float32),
            pltpu.VMEM((B,), jnp.float32),
            pltpu.VMEM((CH,), jnp.float32),
            pltpu.VMEM((16,), jnp.int32),
            pltpu.VMEM((16, D), jnp.float32),
            pltpu.VMEM((16,), jnp.float32),
            pltpu.VMEM((B,), jnp.int32),
            pltpu.SemaphoreType.DMA,
            pltpu.SemaphoreType.DMA,
            pltpu.VMEM_SHARED((SP, D), jnp.float32),
            pltpu.VMEM_SHARED((NS * CH,), jnp.float32),
        ),
    )
    return f(x, ids3)


def _finish_body(sums_ref, counts_ref, out_ref):
    sm = sums_ref[0, :S] + sums_ref[1, :S]
    ct = counts_ref[0] + counts_ref[1]
    nodes = sm / jnp.maximum(ct, 1.0)
    isn = jnp.isnan(nodes)
    n_ok = jnp.sum(jnp.where(isn, 0.0, 1.0))
    mean_val = jnp.sum(jnp.where(isn, 0.0, nodes)) / n_ok
    out_ref[...] = jnp.where(isn, mean_val, nodes)


def kernel(x, segment_ids):
    ids3 = segment_ids.astype(jnp.int32).reshape(NW, NB, B)
    sums_p, counts_p = _sc_segment_sum(x, ids3)
    counts3 = counts_p.reshape(NC, SP)[:, :S].reshape(NC, S, 1)
    return pl.pallas_call(
        _finish_body,
        out_shape=jax.ShapeDtypeStruct((S, D), jnp.float32),
    )(sums_p, counts3)
